# baseline (speedup 1.0000x reference)
import functools

import numpy as np

import jax
import jax.numpy as jnp
from jax import lax
from jax.experimental import pallas as pl
from jax.experimental.pallas import tpu as pltpu
from jax.experimental.pallas import tpu_sc as plsc

_D = 8192
_CB = 512
_NBLK = _D // _CB
_L = 16

_LANE_NP = np.arange(_L, dtype=np.int32)
_PERM_NP = np.concatenate([_LANE_NP ^ 8, _LANE_NP ^ 4, _LANE_NP ^ 2,
                           _LANE_NP ^ 1])


_sc_mesh = plsc.VectorSubcoreMesh(core_axis_name="c", subcore_axis_name="s",
                                  num_cores=1)


_NS = 16
_CHUNK = _D // _NS


@functools.partial(
    pl.kernel,
    out_type=jax.ShapeDtypeStruct((_L,), jnp.int32),
    mesh=_sc_mesh,
    scratch_types=[
        pltpu.VMEM((_CHUNK,), jnp.float32),
        pltpu.VMEM((4 * _L,), jnp.int32),
        pltpu.VMEM((_L,), jnp.float32),
        pltpu.VMEM((_L,), jnp.int32),
        pltpu.VMEM((_NS * _L,), jnp.float32),
        pltpu.VMEM((_NS * _L,), jnp.int32),
        pltpu.VMEM_SHARED((_NS * _L,), jnp.float32),
        pltpu.VMEM_SHARED((_NS * _L,), jnp.int32),
        pltpu.VMEM((_L,), jnp.int32),
    ],
)
def _sc_argmin(wu_hbm, perm_hbm, out_hbm, wu_v, perm_v, mvv, mii, gf_v,
               gi_v, shf, shi, out_v):
    s = lax.axis_index("s")
    pltpu.sync_copy(wu_hbm.at[pl.ds(s * _CHUNK, _CHUNK)], wu_v)
    pltpu.sync_copy(perm_hbm, perm_v)
    lane = perm_v[pl.ds(0, _L)] ^ 8
    base = s * _CHUNK

    mv = jnp.broadcast_to(jnp.float32(jnp.inf), (_L,))
    mi = jnp.broadcast_to(jnp.int32(0), (_L,))
    for sl in range(_CHUNK // _L):
        v = wu_v[pl.ds(sl * _L, _L)]
        idx = base + (sl * _L + lane)
        pred = v < mv
        mv = jnp.where(pred, v, mv)
        mi = jnp.where(pred, idx, mi)
    mvv[...] = mv
    mii[...] = mi
    pltpu.sync_copy(mvv, shf.at[pl.ds(s * _L, _L)])
    pltpu.sync_copy(mii, shi.at[pl.ds(s * _L, _L)])
    plsc.subcore_barrier()

    @pl.when(s == 0)
    def _():
        pltpu.sync_copy(shf, gf_v)
        pltpu.sync_copy(shi, gi_v)
        mv2 = jnp.broadcast_to(jnp.float32(jnp.inf), (_L,))
        mi2 = jnp.broadcast_to(jnp.int32(0), (_L,))
        for t in range(_NS):
            gv = gf_v[pl.ds(t * _L, _L)]
            gi = gi_v[pl.ds(t * _L, _L)]
            take = (gv < mv2) | ((gv == mv2) & (gi < mi2))
            mv2 = jnp.where(take, gv, mv2)
            mi2 = jnp.where(take, gi, mi2)
        for st in range(4):
            pidx = perm_v[pl.ds(st * _L, _L)]
            gv = mv2.at[pidx].get(mode="promise_in_bounds",
                                  unique_indices=True)
            gi = mi2.at[pidx].get(mode="promise_in_bounds",
                                  unique_indices=True)
            take = (gv < mv2) | ((gv == mv2) & (gi < mi2))
            mv2 = jnp.where(take, gv, mv2)
            mi2 = jnp.where(take, gi, mi2)
        out_v[...] = mi2
        pltpu.sync_copy(out_v, out_hbm)


def _flash_body(k_ref, mk_ref, acc_out_ref, sim_out_ref, ml_out_ref,
                acc_ref, m_ref, l_ref, nk_ref):
    j = pl.program_id(0)

    @pl.when(j == 0)
    def _init():
        kv = k_ref[...]
        nk_ref[0, 0] = jnp.sqrt(jnp.sum(kv * kv))
        m_ref[0, 0] = -jnp.inf
        l_ref[0, 0] = 0.0

    blk = mk_ref[...]
    kv = k_ref[...]
    cs = jnp.sum(blk * blk, axis=0, keepdims=True)
    dt = lax.dot_general(kv, blk, (((1,), (0,)), ((), ())),
                         preferred_element_type=jnp.float32)
    sim = dt / (nk_ref[0, 0] * jnp.sqrt(cs))
    sim_out_ref[...] = sim
    m_old = m_ref[0, 0]
    m_new = jnp.maximum(m_old, jnp.max(sim))
    p = jnp.exp(sim - m_new)
    scale = jnp.exp(m_old - m_new)
    l_ref[0, 0] = l_ref[0, 0] * scale + jnp.sum(p)
    m_ref[0, 0] = m_new
    contrib = lax.dot_general(blk, p, (((1,), (1,)), ((), ())),
                              preferred_element_type=jnp.float32)

    @pl.when(j == 0)
    def _first():
        acc_ref[...] = contrib

    @pl.when(j > 0)
    def _rest():
        acc_ref[...] = acc_ref[...] * scale + contrib

    @pl.when(j == _NBLK - 1)
    def _fin():
        acc_out_ref[...] = acc_ref[...]
        ml_out_ref[0, 0] = m_ref[0, 0]
        ml_out_ref[0, 1] = l_ref[0, 0]


def _fix_body(mi_smem, acc_ref, colb_ref, simb_ref, ml_ref, out_ref):
    m = ml_ref[0, 0]
    l = ml_ref[0, 1]
    lm = mi_smem[0] % 128
    lanes = lax.broadcasted_iota(jnp.int32, (1, 128), 1)
    simsel = jnp.sum(jnp.where(lanes == lm, simb_ref[...], 0.0))
    pmi = jnp.exp(simsel - m)
    colv = jnp.sum(jnp.where(lanes == lm, colb_ref[...], 0.0), axis=1,
                   keepdims=True)
    out_ref[...] = (acc_ref[...] - pmi * colv) / l


def kernel(k, u, memory_knowledge, memory_understanding, w_w, w_u, w_lu,
           beta_param):
    mi_vec = _sc_argmin(w_u, jnp.asarray(_PERM_NP))
    k2 = k.reshape(1, _D)
    acc, sims, ml = pl.pallas_call(
        _flash_body,
        grid=(_NBLK,),
        in_specs=[
            pl.BlockSpec((1, _D), lambda j: (0, 0)),
            pl.BlockSpec((_D, _CB), lambda j: (0, j)),
        ],
        out_specs=[
            pl.BlockSpec((_D, 1), lambda j: (0, 0)),
            pl.BlockSpec((1, _CB), lambda j: (0, j)),
            pl.BlockSpec(memory_space=pltpu.SMEM),
        ],
        out_shape=[
            jax.ShapeDtypeStruct((_D, 1), jnp.float32),
            jax.ShapeDtypeStruct((1, _D), jnp.float32),
            jax.ShapeDtypeStruct((1, 2), jnp.float32),
        ],
        scratch_shapes=[
            pltpu.VMEM((_D, 1), jnp.float32),
            pltpu.SMEM((1, 1), jnp.float32),
            pltpu.SMEM((1, 1), jnp.float32),
            pltpu.SMEM((1, 1), jnp.float32),
        ],
        compiler_params=pltpu.CompilerParams(
            dimension_semantics=("arbitrary",),
        ),
    )(k2, memory_knowledge)
    out = pl.pallas_call(
        _fix_body,
        grid_spec=pltpu.PrefetchScalarGridSpec(
            num_scalar_prefetch=1,
            grid=(1,),
            in_specs=[
                pl.BlockSpec((_D, 1), lambda i, mi: (0, 0)),
                pl.BlockSpec((_D, 128), lambda i, mi: (0, mi[0] // 128)),
                pl.BlockSpec((1, 128), lambda i, mi: (0, mi[0] // 128)),
                pl.BlockSpec(memory_space=pltpu.SMEM),
            ],
            out_specs=pl.BlockSpec((_D, 1), lambda i, mi: (0, 0)),
        ),
        out_shape=jax.ShapeDtypeStruct((_D, 1), jnp.float32),
    )(mi_vec, acc, memory_knowledge, sims, ml)
    return out.reshape(1, _D)


# speedup vs baseline: 1.0146x; 1.0146x over previous
import functools

import numpy as np

import jax
import jax.numpy as jnp
from jax import lax
from jax.experimental import pallas as pl
from jax.experimental.pallas import tpu as pltpu
from jax.experimental.pallas import tpu_sc as plsc

_D = 8192
_CB = 512
_NBLK = _D // _CB
_L = 16

_LANE_NP = np.arange(_L, dtype=np.int32)
_PERM_NP = np.concatenate([_LANE_NP ^ 8, _LANE_NP ^ 4, _LANE_NP ^ 2,
                           _LANE_NP ^ 1])


_sc_mesh = plsc.VectorSubcoreMesh(core_axis_name="c", subcore_axis_name="s",
                                  num_cores=1)


_NS = 16
_CHUNK = _D // _NS


@functools.partial(
    pl.kernel,
    out_type=jax.ShapeDtypeStruct((_L,), jnp.int32),
    mesh=_sc_mesh,
    scratch_types=[
        pltpu.VMEM((_CHUNK,), jnp.float32),
        pltpu.VMEM((4 * _L,), jnp.int32),
        pltpu.VMEM((_L,), jnp.float32),
        pltpu.VMEM((_L,), jnp.int32),
        pltpu.VMEM((_NS * _L,), jnp.float32),
        pltpu.VMEM((_NS * _L,), jnp.int32),
        pltpu.VMEM_SHARED((_NS * _L,), jnp.float32),
        pltpu.VMEM_SHARED((_NS * _L,), jnp.int32),
        pltpu.VMEM((_L,), jnp.int32),
    ],
)
def _sc_argmin(wu_hbm, perm_hbm, out_hbm, wu_v, perm_v, mvv, mii, gf_v,
               gi_v, shf, shi, out_v):
    s = lax.axis_index("s")
    pltpu.sync_copy(wu_hbm.at[pl.ds(s * _CHUNK, _CHUNK)], wu_v)
    pltpu.sync_copy(perm_hbm, perm_v)
    lane = perm_v[pl.ds(0, _L)] ^ 8
    base = s * _CHUNK

    mv = jnp.broadcast_to(jnp.float32(jnp.inf), (_L,))
    mi = jnp.broadcast_to(jnp.int32(0), (_L,))
    for sl in range(_CHUNK // _L):
        v = wu_v[pl.ds(sl * _L, _L)]
        idx = base + (sl * _L + lane)
        pred = v < mv
        mv = jnp.where(pred, v, mv)
        mi = jnp.where(pred, idx, mi)
    mvv[...] = mv
    mii[...] = mi
    pltpu.sync_copy(mvv, shf.at[pl.ds(s * _L, _L)])
    pltpu.sync_copy(mii, shi.at[pl.ds(s * _L, _L)])
    plsc.subcore_barrier()

    @pl.when(s == 0)
    def _():
        pltpu.sync_copy(shf, gf_v)
        pltpu.sync_copy(shi, gi_v)
        mv2 = jnp.broadcast_to(jnp.float32(jnp.inf), (_L,))
        mi2 = jnp.broadcast_to(jnp.int32(0), (_L,))
        for t in range(_NS):
            gv = gf_v[pl.ds(t * _L, _L)]
            gi = gi_v[pl.ds(t * _L, _L)]
            take = (gv < mv2) | ((gv == mv2) & (gi < mi2))
            mv2 = jnp.where(take, gv, mv2)
            mi2 = jnp.where(take, gi, mi2)
        for st in range(4):
            pidx = perm_v[pl.ds(st * _L, _L)]
            gv = mv2.at[pidx].get(mode="promise_in_bounds",
                                  unique_indices=True)
            gi = mi2.at[pidx].get(mode="promise_in_bounds",
                                  unique_indices=True)
            take = (gv < mv2) | ((gv == mv2) & (gi < mi2))
            mv2 = jnp.where(take, gv, mv2)
            mi2 = jnp.where(take, gi, mi2)
        out_v[...] = mi2
        pltpu.sync_copy(out_v, out_hbm)


def _flash_body(mi_ref, k_ref, mk_ref, out_ref, acc_ref, m_ref, l_ref,
                nk_ref):
    j = pl.program_id(0)

    @pl.when(j == 0)
    def _init():
        kv = k_ref[...]
        nk_ref[0, 0] = jnp.sqrt(jnp.sum(kv * kv))
        m_ref[0, 0] = -jnp.inf
        l_ref[0, 0] = 0.0

    blk = mk_ref[...]
    kv = k_ref[...]
    cs = jnp.sum(blk * blk, axis=0, keepdims=True)
    dt = lax.dot_general(kv, blk, (((1,), (0,)), ((), ())),
                         preferred_element_type=jnp.float32)
    sim = dt / (nk_ref[0, 0] * jnp.sqrt(cs))
    m_old = m_ref[0, 0]
    m_new = jnp.maximum(m_old, jnp.max(sim))
    p = jnp.exp(sim - m_new)
    scale = jnp.exp(m_old - m_new)
    l_ref[0, 0] = l_ref[0, 0] * scale + jnp.sum(p)
    m_ref[0, 0] = m_new
    col = j * _CB + lax.broadcasted_iota(jnp.int32, (1, _CB), 1)
    pz = jnp.where(col == mi_ref[0], 0.0, p)
    contrib = lax.dot_general(blk, pz, (((1,), (1,)), ((), ())),
                              preferred_element_type=jnp.float32)

    @pl.when(j == 0)
    def _first():
        acc_ref[...] = contrib

    @pl.when(j > 0)
    def _rest():
        acc_ref[...] = acc_ref[...] * scale + contrib

    @pl.when(j == _NBLK - 1)
    def _fin():
        out_ref[...] = acc_ref[...] / l_ref[0, 0]


def kernel(k, u, memory_knowledge, memory_understanding, w_w, w_u, w_lu,
           beta_param):
    mi_vec = _sc_argmin(w_u, jnp.asarray(_PERM_NP))
    k2 = k.reshape(1, _D)
    out = pl.pallas_call(
        _flash_body,
        grid=(_NBLK,),
        in_specs=[
            pl.BlockSpec(memory_space=pltpu.SMEM),
            pl.BlockSpec((1, _D), lambda j: (0, 0)),
            pl.BlockSpec((_D, _CB), lambda j: (0, j)),
        ],
        out_specs=pl.BlockSpec((_D, 1), lambda j: (0, 0)),
        out_shape=jax.ShapeDtypeStruct((_D, 1), jnp.float32),
        scratch_shapes=[
            pltpu.VMEM((_D, 1), jnp.float32),
            pltpu.SMEM((1, 1), jnp.float32),
            pltpu.SMEM((1, 1), jnp.float32),
            pltpu.SMEM((1, 1), jnp.float32),
        ],
        compiler_params=pltpu.CompilerParams(
            dimension_semantics=("arbitrary",),
        ),
    )(mi_vec, k2, memory_knowledge)
    return out.reshape(1, _D)
